# baseline (speedup 1.0000x reference)
import functools

import jax
import jax.numpy as jnp
from jax import lax
from jax.experimental import pallas as pl
from jax.experimental.pallas import tpu as pltpu
from jax.experimental.pallas import tpu_sc as plsc

S = 64
D = 9
C = 4
B = 16
N = 16384
HALVES = 2
XH = S // HALVES
CELLS = XH * S
SENT = CELLS
CNT_W = 2064
PLANES = D * C
PB = 128
NB = N // PB
JCHUNK = 16
NCHUNKS = NB // JCHUNK

_mesh = functools.partial(
    plsc.VectorSubcoreMesh, core_axis_name="c", subcore_axis_name="s"
)


def _sc_body(data_hbm, out_hbm, buf0_v, buf1_v, out_v, cnt_v, in_sem):
  h = lax.axis_index("c")
  b = lax.axis_index("s")

  zf = jnp.zeros((16,), jnp.float32)
  zi = jnp.zeros((16,), jnp.int32)

  def zero_out(i, carry):
    k = i >> 7
    p0 = (i & 127) * 16
    out_v[k, pl.ds(p0, 16)] = zf
    return carry

  lax.fori_loop(0, PLANES * 128, zero_out, 0, unroll=8)

  def zero_cnt(i, carry):
    cnt_v[pl.ds(i * 16, 16)] = zi
    return carry

  lax.fori_loop(0, CNT_W // 16, zero_cnt, 0, unroll=8)

  cwords = JCHUNK * C * PB
  bufs = (buf0_v, buf1_v)

  def chunk_copy(ci, slot):
    return pltpu.make_async_copy(
        data_hbm.at[pl.ds(b * (NB * C * PB) + ci * cwords, cwords)],
        bufs[slot],
        in_sem,
    )

  chunk_copy(0, 0).start()

  for ci in range(NCHUNKS):
    slot = ci % 2
    chunk_copy(ci, slot).wait()
    if ci + 1 < NCHUNKS:
      chunk_copy(ci + 1, 1 - slot).start()
    buf = bufs[slot]

    def block(jj, carry):
      jbase = jj * (C * PB)
      for pp in range(PB // 16):
        p0 = pp * 16
        r0 = buf[pl.ds(jbase + p0, 16)]
        r1 = buf[pl.ds(jbase + PB + p0, 16)]
        r2 = buf[pl.ds(jbase + 2 * PB + p0, 16)]
        r3 = buf[pl.ds(jbase + 3 * PB + p0, 16)]
        d0 = r0 * float(S)
        d1 = r1 * float(S)
        xi = d0.astype(jnp.int32)
        yi = d1.astype(jnp.int32)
        f0 = d0 - xi.astype(jnp.float32)
        f1 = d1 - yi.astype(jnp.float32)
        nz = (r0 != 0.0) | (r1 != 0.0) | (r2 != 0.0) | (r3 != 0.0)
        mine = (xi >> 5) == h
        xl = xi & (XH - 1)
        lcell = xl * S + yi
        ceff = jnp.where(nz & mine, lcell, SENT)
        cnt = plsc.load_gather(cnt_v, [ceff])
        dup, last = plsc.scan_count(ceff)
        rank = cnt + dup - 1
        plsc.store_scatter(cnt_v, [ceff], rank + 1, mask=last)
        ok = (ceff != SENT) & (rank < D)
        rc4 = jnp.minimum(rank, D - 1) * C
        plsc.store_scatter(out_v, [rc4, lcell], f0, mask=ok)
        plsc.store_scatter(out_v, [rc4 + 1, lcell], f1, mask=ok)
        plsc.store_scatter(out_v, [rc4 + 2, lcell], r2, mask=ok)
        plsc.store_scatter(out_v, [rc4 + 3, lcell], r3, mask=ok)
      return carry

    lax.fori_loop(0, JCHUNK, block, 0, unroll=2)

  pltpu.sync_copy(out_v, out_hbm.at[b, :, pl.ds(h * CELLS, CELLS)])


@jax.jit
def kernel(data):
  launch = pl.kernel(
      _sc_body,
      out_type=jax.ShapeDtypeStruct((B, PLANES, S * S), jnp.float32),
      mesh=_mesh(),
      scratch_types=[
          pltpu.VMEM((JCHUNK * C * PB,), jnp.float32),
          pltpu.VMEM((JCHUNK * C * PB,), jnp.float32),
          pltpu.VMEM((PLANES, XH * S), jnp.float32),
          pltpu.VMEM((CNT_W,), jnp.int32),
          pltpu.SemaphoreType.DMA,
      ],
      compiler_params=pltpu.CompilerParams(needs_layout_passes=False),
  )
  dt = data.reshape(B, NB, PB, C).transpose(0, 1, 3, 2).reshape(B * N * C)
  planar = launch(dt).reshape(B, PLANES, S, S)
  return jnp.transpose(planar, (0, 2, 3, 1))


# speedup vs baseline: 1.7012x; 1.7012x over previous
import functools

import jax
import jax.numpy as jnp
from jax import lax
from jax.experimental import pallas as pl
from jax.experimental.pallas import tpu as pltpu
from jax.experimental.pallas import tpu_sc as plsc

S = 64
D = 9
C = 4
B = 16
N = 16384
HALVES = 2
XH = S // HALVES
CELLS = XH * S
SENT = CELLS
CNT_W = 2064
PLANES = D * C
PB = 128
NB = N // PB
JCHUNK = 16
NCHUNKS = NB // JCHUNK

_mesh = functools.partial(
    plsc.VectorSubcoreMesh, core_axis_name="c", subcore_axis_name="s"
)


def _sc_body(data_hbm, out_hbm, buf0_v, buf1_v, out_v, cnt_v, in_sem):
  h = lax.axis_index("c")
  b = lax.axis_index("s")

  zf = jnp.zeros((16,), jnp.float32)
  zi = jnp.zeros((16,), jnp.int32)

  cwords = JCHUNK * C * PB
  bufs = (buf0_v, buf1_v)

  def chunk_copy(ci, slot):
    return pltpu.make_async_copy(
        data_hbm.at[pl.ds(b * (NB * C * PB) + ci * cwords, cwords)],
        bufs[slot],
        in_sem,
    )

  chunk_copy(0, 0).start()

  def zero_out(i, carry):
    k = i >> 7
    p0 = (i & 127) * 16
    out_v[k, pl.ds(p0, 16)] = zf
    return carry

  lax.fori_loop(0, PLANES * 128, zero_out, 0, unroll=8)

  def zero_cnt(i, carry):
    cnt_v[pl.ds(i * 16, 16)] = zi
    return carry

  lax.fori_loop(0, CNT_W // 16, zero_cnt, 0, unroll=8)

  for ci in range(NCHUNKS):
    slot = ci % 2
    chunk_copy(ci, slot).wait()
    if ci + 1 < NCHUNKS:
      chunk_copy(ci + 1, 1 - slot).start()
    buf = bufs[slot]

    def stage_a(g):
      p0 = (g >> 3) * (C * PB) + (g & 7) * 16
      r0 = buf[pl.ds(p0, 16)]
      r1 = buf[pl.ds(p0 + PB, 16)]
      r2 = buf[pl.ds(p0 + 2 * PB, 16)]
      r3 = buf[pl.ds(p0 + 3 * PB, 16)]
      d0 = r0 * float(S)
      d1 = r1 * float(S)
      xi = d0.astype(jnp.int32)
      yi = d1.astype(jnp.int32)
      f0 = d0 - xi.astype(jnp.float32)
      f1 = d1 - yi.astype(jnp.float32)
      nz = (r0 != 0.0) | (r1 != 0.0) | (r2 != 0.0) | (r3 != 0.0)
      mine = (xi >> 5) == h
      xl = xi & (XH - 1)
      lcell = xl * S + yi
      ceff = jnp.where(nz & mine, lcell, SENT)
      dup, last = plsc.scan_count(ceff)
      return (ceff, lcell, dup, last, f0, f1, r2, r3)

    def stage_b(st):
      ceff, lcell, dup, last, f0, f1, r2, r3 = st
      cnt = plsc.load_gather(cnt_v, [ceff])
      rank = cnt + dup - 1
      plsc.store_scatter(cnt_v, [ceff], cnt + dup, mask=last)
      ok = (ceff != SENT) & (rank < D)
      rc4 = jnp.minimum(rank, D - 1) * C
      plsc.store_scatter(out_v, [rc4, lcell], f0, mask=ok)
      plsc.store_scatter(out_v, [rc4 + 1, lcell], f1, mask=ok)
      plsc.store_scatter(out_v, [rc4 + 2, lcell], r2, mask=ok)
      plsc.store_scatter(out_v, [rc4 + 3, lcell], r3, mask=ok)

    def steady(g, carry):
      st0, st1, st2 = carry
      nxt = stage_a(g)
      stage_b(st0)
      return (st1, st2, nxt)

    init = (stage_a(0), stage_a(1), stage_a(2))
    st0, st1, st2 = lax.fori_loop(
        3, JCHUNK * (PB // 16), steady, init, unroll=8
    )
    stage_b(st0)
    stage_b(st1)
    stage_b(st2)

  pltpu.sync_copy(out_v, out_hbm.at[b, :, pl.ds(h * CELLS, CELLS)])


@jax.jit
def kernel(data):
  launch = pl.kernel(
      _sc_body,
      out_type=jax.ShapeDtypeStruct((B, PLANES, S * S), jnp.float32),
      mesh=_mesh(),
      scratch_types=[
          pltpu.VMEM((JCHUNK * C * PB,), jnp.float32),
          pltpu.VMEM((JCHUNK * C * PB,), jnp.float32),
          pltpu.VMEM((PLANES, XH * S), jnp.float32),
          pltpu.VMEM((CNT_W,), jnp.int32),
          pltpu.SemaphoreType.DMA,
      ],
      compiler_params=pltpu.CompilerParams(needs_layout_passes=False),
  )
  dt = data.reshape(B, NB, PB, C).transpose(0, 1, 3, 2).reshape(B * N * C)
  flat = launch(dt)
  return lax.reshape(flat, (B, S, S, PLANES), dimensions=(0, 2, 1))


# speedup vs baseline: 1.7601x; 1.0346x over previous
import functools

import jax
import jax.numpy as jnp
from jax import lax
from jax.experimental import pallas as pl
from jax.experimental.pallas import tpu as pltpu
from jax.experimental.pallas import tpu_sc as plsc

S = 64
D = 9
C = 4
B = 16
N = 16384
HALVES = 2
XH = S // HALVES
CELLS = XH * S
SENT = CELLS
CNT_W = 2064
PLANES = D * C
PB = 128
NB = N // PB
JCHUNK = 16
NCHUNKS = NB // JCHUNK

_mesh = functools.partial(
    plsc.VectorSubcoreMesh, core_axis_name="c", subcore_axis_name="s"
)


def _sc_body(data_hbm, out_hbm, buf0_v, buf1_v, out_v, cnt_v, in_sem):
  h = lax.axis_index("c")
  b = lax.axis_index("s")

  zf = jnp.zeros((16,), jnp.float32)
  zi = jnp.zeros((16,), jnp.int32)

  cwords = JCHUNK * C * PB
  bufs = (buf0_v, buf1_v)

  def chunk_copy(ci, slot):
    return pltpu.make_async_copy(
        data_hbm.at[pl.ds(b * (NB * C * PB) + ci * cwords, cwords)],
        bufs[slot],
        in_sem,
    )

  chunk_copy(0, 0).start()

  def zero_out(i, carry):
    k = i >> 7
    p0 = (i & 127) * 16
    out_v[k, pl.ds(p0, 16)] = zf
    return carry

  lax.fori_loop(0, PLANES * 128, zero_out, 0, unroll=8)

  def zero_cnt(i, carry):
    cnt_v[pl.ds(i * 16, 16)] = zi
    return carry

  lax.fori_loop(0, CNT_W // 16, zero_cnt, 0, unroll=8)

  for ci in range(NCHUNKS):
    slot = ci % 2
    chunk_copy(ci, slot).wait()
    if ci + 1 < NCHUNKS:
      chunk_copy(ci + 1, 1 - slot).start()
    buf = bufs[slot]

    def stage_a(g):
      p0 = (g >> 3) * (C * PB) + (g & 7) * 16
      r0 = buf[pl.ds(p0, 16)]
      r1 = buf[pl.ds(p0 + PB, 16)]
      r2 = buf[pl.ds(p0 + 2 * PB, 16)]
      r3 = buf[pl.ds(p0 + 3 * PB, 16)]
      d0 = r0 * float(S)
      d1 = r1 * float(S)
      xi = d0.astype(jnp.int32)
      yi = d1.astype(jnp.int32)
      f0 = d0 - xi.astype(jnp.float32)
      f1 = d1 - yi.astype(jnp.float32)
      nz = (r0 != 0.0) | (r1 != 0.0) | (r2 != 0.0) | (r3 != 0.0)
      mine = (xi >> 5) == h
      xl = xi & (XH - 1)
      lcell = xl * S + yi
      ceff = jnp.where(nz & mine, lcell, SENT)
      dup, last = plsc.scan_count(ceff)
      return (ceff, lcell, dup, last, f0, f1, r2, r3)

    def stage_b(st):
      ceff, lcell, dup, last, f0, f1, r2, r3 = st
      cnt = plsc.load_gather(cnt_v, [ceff])
      rank = cnt + dup - 1
      plsc.store_scatter(cnt_v, [ceff], cnt + dup, mask=last)
      ok = (ceff != SENT) & (rank < D)
      rc4 = jnp.minimum(rank, D - 1) * C
      plsc.store_scatter(out_v, [rc4, lcell], f0, mask=ok)
      plsc.store_scatter(out_v, [rc4 + 1, lcell], f1, mask=ok)
      plsc.store_scatter(out_v, [rc4 + 2, lcell], r2, mask=ok)
      plsc.store_scatter(out_v, [rc4 + 3, lcell], r3, mask=ok)

    def steady(g, carry):
      st0, st1, st2 = carry
      nxt = stage_a(g)
      stage_b(st0)
      return (st1, st2, nxt)

    init = (stage_a(0), stage_a(1), stage_a(2))
    st0, st1, st2 = lax.fori_loop(
        3, JCHUNK * (PB // 16), steady, init, unroll=4
    )
    stage_b(st0)
    stage_b(st1)
    stage_b(st2)

  pltpu.sync_copy(out_v, out_hbm.at[b, :, pl.ds(h * CELLS, CELLS)])


@jax.jit
def kernel(data):
  launch = pl.kernel(
      _sc_body,
      out_type=jax.ShapeDtypeStruct((B, PLANES, S * S), jnp.float32),
      mesh=_mesh(),
      scratch_types=[
          pltpu.VMEM((JCHUNK * C * PB,), jnp.float32),
          pltpu.VMEM((JCHUNK * C * PB,), jnp.float32),
          pltpu.VMEM((PLANES, XH * S), jnp.float32),
          pltpu.VMEM((CNT_W,), jnp.int32),
          pltpu.SemaphoreType.DMA,
      ],
      compiler_params=pltpu.CompilerParams(needs_layout_passes=False),
  )
  dt = data.reshape(B, NB, PB, C).transpose(0, 1, 3, 2).reshape(B * N * C)
  flat = launch(dt)
  return lax.reshape(flat, (B, S, S, PLANES), dimensions=(0, 2, 1))
